# baseline (speedup 1.0000x reference)
import jax
import jax.numpy as jnp
from jax.experimental import pallas as pl
from jax.experimental.pallas import tpu as pltpu

QUEUE_ROWS = 32768
SHIFT = 4096
CHUNKS = (
    [(256, False), (256, False), (512, False), (1024, False), (2048, False)]
    + [(4096, False)] * 6
    + [(2048, True), (1024, True), (512, True), (256, True), (128, True), (128, True)]
)
N_CHUNKS = len(CHUNKS)


def _fifo_copy(x_ref, q_ref, o_ref, buf, sin, sout):
    ins = []
    outs = []
    out_off = 0
    x_off = 0
    for c, (rows, from_x) in enumerate(CHUNKS):
        if from_x:
            src = x_ref.at[pl.ds(x_off, rows)]
            x_off += rows
        else:
            src = q_ref.at[pl.ds(SHIFT + out_off, rows)]
        stage = buf.at[pl.ds(out_off, rows)]
        ins.append(pltpu.make_async_copy(src, stage, sin.at[c]))
        outs.append(pltpu.make_async_copy(
            stage, o_ref.at[pl.ds(out_off, rows)], sout.at[c]))
        out_off += rows

    for c in range(N_CHUNKS):
        ins[c].start()
    for c in range(N_CHUNKS):
        ins[c].wait()
        outs[c].start()
    for c in range(N_CHUNKS):
        outs[c].wait()


def kernel(x, queue):
    return pl.pallas_call(
        _fifo_copy,
        out_shape=jax.ShapeDtypeStruct(queue.shape, queue.dtype),
        in_specs=[
            pl.BlockSpec(memory_space=pl.ANY),
            pl.BlockSpec(memory_space=pl.ANY),
        ],
        out_specs=pl.BlockSpec(memory_space=pl.ANY),
        scratch_shapes=[
            pltpu.VMEM((QUEUE_ROWS, 128), jnp.float32),
            pltpu.SemaphoreType.DMA((N_CHUNKS,)),
            pltpu.SemaphoreType.DMA((N_CHUNKS,)),
        ],
    )(x, queue)
